# Initial kernel scaffold; baseline (speedup 1.0000x reference)
import jax
import jax.numpy as jnp
from jax.experimental import pallas as pl


def kernel(A_list, Nodes_list, nodes_mask_list, edge_weights, p1_scorer, p1_Wu, p1_Uu, p1_bu, p1_Wr, p1_Ur, p1_br, p1_Wh, p1_Uh, p1_bh, p1_Q0, p2_scorer, p2_Wu, p2_Uu, p2_bu, p2_Wr, p2_Ur, p2_br, p2_Wh, p2_Uh, p2_bh, p2_Q0):
    raise NotImplementedError("write your pallas kernel here")


# speedup vs baseline: 4.5104x; 4.5104x over previous
import functools

import jax
import jax.numpy as jnp
from jax import lax
from jax.experimental import pallas as pl
from jax.experimental.pallas import tpu as pltpu
from jax.experimental.pallas import tpu_sc as plsc

T, N, E, D = 3, 10000, 320000, 128
NC, NS = 2, 16
NW = NC * NS
EW = E // NW
CH = 80
NCHUNK = EW // CH
NG = 5
SG = NCHUNK // NG
N_PAD = 10240
RPS = N_PAD // NS
RZ = 128


def _scores_body(x_ref, scorer_ref, mask_ref, o_ref):
    sc = scorer_ref[...]
    nrm = jnp.sqrt(jnp.sum(sc * sc))
    s = jnp.dot(x_ref[0], sc, preferred_element_type=jnp.float32)
    o_ref[0] = s / nrm + mask_ref[0]


def _scores_call(x, scorer, mask):
    nt = x.shape[0]
    return pl.pallas_call(
        _scores_body,
        grid=(nt,),
        in_specs=[
            pl.BlockSpec((1, N, D), lambda t: (t, 0, 0)),
            pl.BlockSpec((D, 1), lambda t: (0, 0)),
            pl.BlockSpec((1, N, 1), lambda t: (t, 0, 0)),
        ],
        out_specs=pl.BlockSpec((1, N, 1), lambda t: (t, 0, 0)),
        out_shape=jax.ShapeDtypeStruct((nt, N, 1), jnp.float32),
    )(x, scorer, mask)


def _gru_body(rows_ref, vals_ref, wu, uu, bu, wr, ur, br, wh, uh, bh, q0,
              qs_ref):
    q = q0[...]
    for t in range(T):
        rs = rows_ref[t] * jnp.tanh(vals_ref[t])

        def wz(m):
            return lax.dot_general(m[...], rs, (((1,), (1,)), ((), ())),
                                   preferred_element_type=jnp.float32)

        upd = jax.nn.sigmoid(wz(wu) + jnp.dot(uu[...], q) + bu[...])
        rst = jax.nn.sigmoid(wz(wr) + jnp.dot(ur[...], q) + br[...])
        hcap = jnp.tanh(wz(wh) + jnp.dot(uh[...], rst * q) + bh[...])
        q = (1.0 - upd) * q + upd * hcap
        qs_ref[t] = q


def _gru_call(rows, vals, p):
    return pl.pallas_call(
        _gru_body,
        out_shape=jax.ShapeDtypeStruct((T, D, D), jnp.float32),
    )(rows, vals, p["Wu"], p["Uu"], p["bu"], p["Wr"], p["Ur"], p["br"],
      p["Wh"], p["Uh"], p["bh"], p["Q0"])


def _support_body(x_ref, q_ref, o_ref):
    o_ref[0] = jnp.dot(x_ref[0], q_ref[0],
                       preferred_element_type=jnp.float32)


def _support_call(x, qs):
    nt = x.shape[0]
    return pl.pallas_call(
        _support_body,
        grid=(nt,),
        in_specs=[
            pl.BlockSpec((1, N, D), lambda t: (t, 0, 0)),
            pl.BlockSpec((1, D, D), lambda t: (t, 0, 0)),
        ],
        out_specs=pl.BlockSpec((1, N, D), lambda t: (t, 0, 0)),
        out_shape=jax.ShapeDtypeStruct((nt, N, D), jnp.float32),
    )(x, qs)


def _comb_body(p_ref, o_ref):
    o_ref[0] = jnp.maximum(p_ref[0, 0, :N] + p_ref[0, 1, :N], 0.0)


def _comb_call(partials):
    nt = partials.shape[0]
    return pl.pallas_call(
        _comb_body,
        grid=(nt,),
        in_specs=[
            pl.BlockSpec((1, NC, N_PAD, D), lambda t: (t, 0, 0, 0)),
        ],
        out_specs=pl.BlockSpec((1, N, D), lambda t: (t, 0, 0)),
        out_shape=jax.ShapeDtypeStruct((nt, N, D), jnp.float32),
    )(partials)


def _bcast_lane(v16, lane):
    idx = jnp.full((16, 1), lane, jnp.int32)
    return lax.gather(
        v16, idx,
        dimension_numbers=lax.GatherDimensionNumbers(
            offset_dims=(), collapsed_slice_dims=(0,), start_index_map=(0,)),
        slice_sizes=(1,),
        mode=lax.GatherScatterMode.PROMISE_IN_BOUNDS)


def _mp_body(nst, x_hbm, src_hbm, dst_hbm, ew_hbm, zeros_hbm, out_hbm,
             idx_g, dst_g, ew_g, rows_v, acc_sh, sem):
    c = lax.axis_index("c")
    s = lax.axis_index("s")
    wid = s * NC + c
    for st in range(nst):
        pltpu.sync_copy(zeros_hbm.at[pl.ds(0, CH)], rows_v)

        def zero_body(z, carry):
            pltpu.sync_copy(rows_v, acc_sh.at[pl.ds(s * RPS + z * CH, CH)])
            return carry

        lax.fori_loop(0, RPS // CH, zero_body, 0)
        plsc.subcore_barrier()

        def group_body(g, carry):
            pltpu.sync_copy(src_hbm.at[st, wid, g], idx_g)
            pltpu.sync_copy(dst_hbm.at[st, wid, g], dst_g)
            pltpu.sync_copy(ew_hbm.at[st, wid, g], ew_g)

            def chunk_body(i, carry2):
                pltpu.async_copy(x_hbm.at[st].at[idx_g.at[i]], rows_v,
                                 sem).wait()
                for e16 in range(CH // 16):
                    v16 = ew_g[i, pl.ds(e16 * 16, 16)]
                    for lane in range(16):
                        w = _bcast_lane(v16, lane)
                        e = e16 * 16 + lane
                        for j in range(D // 16):
                            sl = pl.ds(j * 16, 16)
                            rows_v[e, sl] = rows_v[e, sl] * w
                pltpu.sync_copy(rows_v, acc_sh.at[dst_g.at[i]], add=True)
                return carry2

            lax.fori_loop(0, SG, chunk_body, 0)
            return carry

        lax.fori_loop(0, NG, group_body, 0)
        plsc.subcore_barrier()
        pltpu.sync_copy(acc_sh.at[pl.ds(s * RPS, RPS)],
                        out_hbm.at[st, c, pl.ds(s * RPS, RPS)])
        plsc.subcore_barrier()


def _mp_call(x, src_r, dst_r, ew_r, zeros_small):
    nst = x.shape[0]
    mesh = plsc.VectorSubcoreMesh(core_axis_name="c", subcore_axis_name="s",
                                  num_cores=NC, num_subcores=NS)
    kern = pl.kernel(
        functools.partial(_mp_body, nst),
        out_type=jax.ShapeDtypeStruct((nst, NC, N_PAD, D), jnp.float32),
        mesh=mesh,
        scratch_types=[
            pltpu.VMEM((SG, CH), jnp.int32),
            pltpu.VMEM((SG, CH), jnp.int32),
            pltpu.VMEM((SG, CH), jnp.float32),
            pltpu.VMEM((CH, D), jnp.float32),
            pltpu.VMEM_SHARED((N_PAD, D), jnp.float32),
            pltpu.SemaphoreType.DMA,
        ],
    )
    return kern(x, src_r, dst_r, ew_r, zeros_small)


def _layer(x, src_r, dst_r, ew_r, mask, zeros_small, p, last_only):
    scores = _scores_call(x, p["scorer"], mask)
    flat = scores.reshape(T, N)
    vals, idx = lax.top_k(flat, D)
    rows = jax.vmap(lambda xt, it: xt[it])(x, idx)
    qs = _gru_call(rows, vals[:, :, None], p)
    if last_only:
        sup = _support_call(x[T - 1:], qs[T - 1:])
        partials = _mp_call(sup, src_r[T - 1:], dst_r[T - 1:],
                            ew_r[T - 1:], zeros_small)
        return _comb_call(partials)
    sup = _support_call(x, qs)
    partials = _mp_call(sup, src_r, dst_r, ew_r, zeros_small)
    return _comb_call(partials)


def kernel(A_list, Nodes_list, nodes_mask_list, edge_weights, p1_scorer,
           p1_Wu, p1_Uu, p1_bu, p1_Wr, p1_Ur, p1_br, p1_Wh, p1_Uh, p1_bh,
           p1_Q0, p2_scorer, p2_Wu, p2_Uu, p2_bu, p2_Wr, p2_Ur, p2_br,
           p2_Wh, p2_Uh, p2_bh, p2_Q0):
    a = A_list.astype(jnp.int32)
    src_r = a[:, 0].reshape(T, NW, NG, SG, CH)
    dst_r = a[:, 1].reshape(T, NW, NG, SG, CH)
    ew_r = edge_weights.astype(jnp.float32).reshape(T, NW, NG, SG, CH)
    zeros_small = jnp.zeros((RZ, D), jnp.float32)
    p1 = {"scorer": p1_scorer, "Wu": p1_Wu, "Uu": p1_Uu, "bu": p1_bu,
          "Wr": p1_Wr, "Ur": p1_Ur, "br": p1_br, "Wh": p1_Wh, "Uh": p1_Uh,
          "bh": p1_bh, "Q0": p1_Q0}
    p2 = {"scorer": p2_scorer, "Wu": p2_Wu, "Uu": p2_Uu, "bu": p2_bu,
          "Wr": p2_Wr, "Ur": p2_Ur, "br": p2_br, "Wh": p2_Wh, "Uh": p2_Uh,
          "bh": p2_bh, "Q0": p2_Q0}
    x2 = _layer(Nodes_list, src_r, dst_r, ew_r, nodes_mask_list,
                zeros_small, p1, last_only=False)
    out = _layer(x2, src_r, dst_r, ew_r, nodes_mask_list, zeros_small, p2,
                 last_only=True)
    return out[0]
